# Initial kernel scaffold; baseline (speedup 1.0000x reference)
import jax
import jax.numpy as jnp
from jax.experimental import pallas as pl


def kernel(nf, ef, edge_index, msg_W0, msg_b0, msg_W1, msg_b1, msg_W2, msg_b2, msg_W3, msg_b3, red_W0, red_b0, red_W1, red_b1, red_W2, red_b2, red_W3, red_b3):
    raise NotImplementedError("write your pallas kernel here")


# speedup vs baseline: 2.7304x; 2.7304x over previous
import functools

import jax
import jax.numpy as jnp
from jax import lax
from jax.experimental import pallas as pl
from jax.experimental.pallas import tpu as pltpu
from jax.experimental.pallas import tpu_sc as plsc

_N = 10000
_E = 320000
_NC = 2
_NS = 16
_NW = _NC * _NS
_BIG = 3.0e38


def _tc_pq(nf, w0ab):
    n = nf.shape[0]
    bn = 2000

    def body(nf_ref, w_ref, pq_ref):
        pq_ref[...] = jnp.dot(nf_ref[...], w_ref[...],
                              preferred_element_type=jnp.float32)

    return pl.pallas_call(
        body,
        grid=(n // bn,),
        in_specs=[
            pl.BlockSpec((bn, 128), lambda i: (i, 0)),
            pl.BlockSpec((128, 128), lambda i: (0, 0)),
        ],
        out_specs=pl.BlockSpec((bn, 128), lambda i: (i, 0)),
        out_shape=jax.ShapeDtypeStruct((n, 128), jnp.float32),
    )(nf, w0ab)


def _sc_gather_add(pq, src, dst):
    e = src.shape[0]
    epw = e // _NW
    cb = 400
    sub = 80
    nchunk = epw // cb
    mesh = plsc.VectorSubcoreMesh(core_axis_name="c", subcore_axis_name="s")

    @functools.partial(
        pl.kernel,
        out_type=jax.ShapeDtypeStruct((e, 64), jnp.float32),
        mesh=mesh,
        scratch_types=[
            pltpu.VMEM((cb,), jnp.int32),
            pltpu.VMEM((cb,), jnp.int32),
            pltpu.VMEM((sub, 128), jnp.float32),
            pltpu.VMEM((sub, 128), jnp.float32),
            pltpu.VMEM((cb, 64), jnp.float32),
            pltpu.SemaphoreType.DMA,
        ],
    )
    def k(pq_hbm, src_hbm, dst_hbm, g_hbm, srcv, dstv, prow, qrow, gv, sem):
        wid = lax.axis_index("s") * _NC + lax.axis_index("c")
        base_w = wid * epw

        @pl.loop(0, nchunk)
        def _chunk(ci):
            base = base_w + ci * cb
            pltpu.sync_copy(src_hbm.at[pl.ds(base, cb)], srcv)
            pltpu.sync_copy(dst_hbm.at[pl.ds(base, cb)], dstv)

            @pl.loop(0, cb // sub)
            def _sub(s):
                off = s * sub
                c1 = pltpu.async_copy(
                    pq_hbm.at[srcv.at[pl.ds(off, sub)]], prow, sem)
                c2 = pltpu.async_copy(
                    pq_hbm.at[dstv.at[pl.ds(off, sub)]], qrow, sem)
                c1.wait()
                c2.wait()

                @pl.loop(0, sub)
                def _row(r):
                    for c in range(4):
                        sl = pl.ds(c * 16, 16)
                        gv[off + r, sl] = prow[r, sl] \
                            + qrow[r, pl.ds(64 + c * 16, 16)]

            pltpu.sync_copy(gv, g_hbm.at[pl.ds(base, cb)])

    return k(pq, src, dst)


def _tc_edge_mlp(g, ef, w0c, b0, w1, b1, w2, b2, w3, b3):
    e = g.shape[0]
    eb = 2000

    def body(g_ref, ef_ref, w0c_ref, b0_ref, w1_ref, b1_ref, w2_ref, b2_ref,
             w3_ref, b3_ref, o_ref):
        h = g_ref[...] + jnp.dot(ef_ref[...], w0c_ref[...],
                                 preferred_element_type=jnp.float32)
        h = jnp.maximum(h + b0_ref[...], 0.0)
        h = jnp.maximum(jnp.dot(h, w1_ref[...],
                                preferred_element_type=jnp.float32)
                        + b1_ref[...], 0.0)
        h = jnp.maximum(jnp.dot(h, w2_ref[...],
                                preferred_element_type=jnp.float32)
                        + b2_ref[...], 0.0)
        t = jnp.dot(h, w3_ref[...], preferred_element_type=jnp.float32) \
            + b3_ref[...]
        kk = jax.nn.sigmoid(t[:, 0:1])
        f12 = t[:, 1:25] * kk
        f3 = t[:, 25:37] * kk
        f4 = t[:, 37:49] * kk
        ones = jnp.ones((eb, 1), jnp.float32)
        z7 = jnp.zeros((eb, 7), jnp.float32)
        z4 = jnp.zeros((eb, 4), jnp.float32)
        z64 = jnp.zeros((eb, 64), jnp.float32)
        o_ref[...] = jnp.concatenate(
            [ones, f12, z7, f3, z4, f4, z4, z64], axis=1)

    wspec = lambda shape: pl.BlockSpec(shape, lambda i: (0, 0))
    return pl.pallas_call(
        body,
        grid=(e // eb,),
        in_specs=[
            pl.BlockSpec((eb, 64), lambda i: (i, 0)),
            pl.BlockSpec((eb, 16), lambda i: (i, 0)),
            wspec((16, 64)), wspec((1, 64)),
            wspec((64, 64)), wspec((1, 64)),
            wspec((64, 64)), wspec((1, 64)),
            wspec((64, 49)), wspec((1, 49)),
        ],
        out_specs=pl.BlockSpec((eb, 128), lambda i: (i, 0)),
        out_shape=jax.ShapeDtypeStruct((e, 128), jnp.float32),
    )(g, ef, w0c, b0, w1, b1, w2, b2, w3, b3)


def _sc_segment_reduce(f, dst):
    e = dst.shape[0]
    npw = 320
    npad = _NW * npw
    dc = 2000
    ndc = e // dc
    pb = 256
    cap = pb + 32
    mesh = plsc.VectorSubcoreMesh(core_axis_name="c", subcore_axis_name="s")

    @functools.partial(
        pl.kernel,
        out_type=jax.ShapeDtypeStruct((npad, 64), jnp.float32),
        mesh=mesh,
        scratch_types=[
            pltpu.VMEM((dc,), jnp.int32),
            pltpu.VMEM((cap,), jnp.int32),
            pltpu.VMEM((cap,), jnp.int32),
            pltpu.VMEM((pb, 128), jnp.float32),
            pltpu.VMEM((npw, 64), jnp.float32),
            pltpu.SemaphoreType.DMA,
        ],
        compiler_params=pltpu.CompilerParams(needs_layout_passes=False),
    )
    def k(f_hbm, dst_hbm, r_hbm, dstv, eidb, dstb, rows, acc, sem):
        wid = lax.axis_index("s") * _NC + lax.axis_index("c")
        lo = wid * npw

        @pl.loop(0, npw)
        def _init(r):
            acc[r, pl.ds(0, 16)] = jnp.zeros((16,), jnp.float32)
            acc[r, pl.ds(16, 16)] = jnp.zeros((16,), jnp.float32)
            acc[r, pl.ds(32, 16)] = jnp.full((16,), _BIG, jnp.float32)
            acc[r, pl.ds(48, 16)] = jnp.full((16,), -_BIG, jnp.float32)

        @pl.loop(0, cap // 16)
        def _initb(i):
            eidb[pl.ds(i * 16, 16)] = jnp.zeros((16,), jnp.int32)

        def flush(m):
            c1 = pltpu.async_copy(
                f_hbm.at[eidb.at[pl.ds(0, 128)]], rows.at[pl.ds(0, 128)], sem)
            c2 = pltpu.async_copy(
                f_hbm.at[eidb.at[pl.ds(128, 128)]], rows.at[pl.ds(128, 128)],
                sem)
            c1.wait()
            c2.wait()

            @pl.loop(0, m)
            def _rmw(j):
                dl = dstb[pl.ds(j, 16)][0] - lo
                acc[dl, pl.ds(0, 16)] = acc[dl, pl.ds(0, 16)] \
                    + rows[j, pl.ds(0, 16)]
                acc[dl, pl.ds(16, 16)] = acc[dl, pl.ds(16, 16)] \
                    + rows[j, pl.ds(16, 16)]
                acc[dl, pl.ds(32, 16)] = jnp.minimum(
                    acc[dl, pl.ds(32, 16)], rows[j, pl.ds(32, 16)])
                acc[dl, pl.ds(48, 16)] = jnp.maximum(
                    acc[dl, pl.ds(48, 16)], rows[j, pl.ds(48, 16)])

        def scan_chunk(ci, cnt):
            pltpu.sync_copy(dst_hbm.at[pl.ds(ci * dc, dc)], dstv)

            def scan16(i, cnt):
                d16 = dstv[pl.ds(i * 16, 16)]
                m = (d16 >= lo) & (d16 < lo + npw)
                eid = ci * dc + i * 16 + lax.iota(jnp.int32, 16)
                plsc.store_compressed(eidb.at[pl.ds(cnt, 16)], eid, mask=m)
                plsc.store_compressed(dstb.at[pl.ds(cnt, 16)], d16, mask=m)
                cnt = cnt + jnp.sum(m.astype(jnp.int32))

                @pl.when(cnt >= pb)
                def _do_flush():
                    flush(pb)
                    eidb[pl.ds(0, 16)] = eidb[pl.ds(pb, 16)]
                    dstb[pl.ds(0, 16)] = dstb[pl.ds(pb, 16)]

                return jnp.where(cnt >= pb, cnt - pb, cnt)

            return lax.fori_loop(0, dc // 16, scan16, cnt)

        cnt = lax.fori_loop(0, ndc, scan_chunk, jnp.int32(0))
        flush(cnt)
        pltpu.sync_copy(acc, r_hbm.at[pl.ds(lo, npw)])

    return k(f, dst)


def _tc_node_mlp(nf, r, w0, b0, w1, b1, w2, b2, w3, b3):
    n = nf.shape[0]
    bn = 2000

    def body(nf_ref, r_ref, w0_ref, b0_ref, w1_ref, b1_ref, w2_ref, b2_ref,
             w3_ref, b3_ref, o_ref):
        rr = r_ref[...]
        cnt = rr[:, 0:1]
        nf1 = rr[:, 1:13] / jnp.maximum(cnt, 1.0)
        nf2 = rr[:, 13:25]
        mask = cnt > 0.0
        nf3 = jnp.where(mask, rr[:, 32:44], 0.0)
        nf4 = jnp.where(mask, rr[:, 48:60], 0.0)
        x = jnp.concatenate([nf_ref[...], nf1, nf2, nf3, nf4], axis=1)
        h = jnp.maximum(jnp.dot(x, w0_ref[...],
                                preferred_element_type=jnp.float32)
                        + b0_ref[...], 0.0)
        h = jnp.maximum(jnp.dot(h, w1_ref[...],
                                preferred_element_type=jnp.float32)
                        + b1_ref[...], 0.0)
        h = jnp.maximum(jnp.dot(h, w2_ref[...],
                                preferred_element_type=jnp.float32)
                        + b2_ref[...], 0.0)
        o_ref[...] = jnp.dot(h, w3_ref[...],
                             preferred_element_type=jnp.float32) + b3_ref[...]

    wspec = lambda shape: pl.BlockSpec(shape, lambda i: (0, 0))
    return pl.pallas_call(
        body,
        grid=(n // bn,),
        in_specs=[
            pl.BlockSpec((bn, 128), lambda i: (i, 0)),
            pl.BlockSpec((bn, 64), lambda i: (i, 0)),
            wspec((176, 64)), wspec((1, 64)),
            wspec((64, 64)), wspec((1, 64)),
            wspec((64, 64)), wspec((1, 64)),
            wspec((64, 128)), wspec((1, 128)),
        ],
        out_specs=pl.BlockSpec((bn, 128), lambda i: (i, 0)),
        out_shape=jax.ShapeDtypeStruct((n, 128), jnp.float32),
    )(nf, r, w0, b0, w1, b1, w2, b2, w3, b3)


def kernel(nf, ef, edge_index, msg_W0, msg_b0, msg_W1, msg_b1, msg_W2, msg_b2,
           msg_W3, msg_b3, red_W0, red_b0, red_W1, red_b1, red_W2, red_b2,
           red_W3, red_b3):
    src = edge_index[0]
    dst = edge_index[1]
    w0a = msg_W0[:128]
    w0b = msg_W0[128:256]
    w0c = msg_W0[256:]

    pq = _tc_pq(nf, jnp.concatenate([w0a, w0b], axis=1))
    g = _sc_gather_add(pq, src, dst)
    f = _tc_edge_mlp(
        g, ef, w0c, msg_b0.reshape(1, 64), msg_W1, msg_b1.reshape(1, 64),
        msg_W2, msg_b2.reshape(1, 64), msg_W3, msg_b3.reshape(1, 49))
    r = _sc_segment_reduce(f, dst)
    out = _tc_node_mlp(
        nf, r, red_W0, red_b0.reshape(1, 64), red_W1, red_b1.reshape(1, 64),
        red_W2, red_b2.reshape(1, 64), red_W3, red_b3.reshape(1, 128))
    return out


# speedup vs baseline: 3.8993x; 1.4281x over previous
import functools

import jax
import jax.numpy as jnp
from jax import lax
from jax.experimental import pallas as pl
from jax.experimental.pallas import tpu as pltpu
from jax.experimental.pallas import tpu_sc as plsc

_N = 10000
_E = 320000
_NC = 2
_NS = 16
_NW = _NC * _NS
_BIG = 3.0e38


def _tc_pq(nf, w0ab):
    n = nf.shape[0]
    bn = 2000

    def body(nf_ref, w_ref, pq_ref):
        pq_ref[...] = jnp.dot(nf_ref[...], w_ref[...],
                              preferred_element_type=jnp.float32)

    return pl.pallas_call(
        body,
        grid=(n // bn,),
        in_specs=[
            pl.BlockSpec((bn, 128), lambda i: (i, 0)),
            pl.BlockSpec((128, 128), lambda i: (0, 0)),
        ],
        out_specs=pl.BlockSpec((bn, 128), lambda i: (i, 0)),
        out_shape=jax.ShapeDtypeStruct((n, 128), jnp.float32),
    )(nf, w0ab)


def _sc_gather_add(pq, src, dst):
    e = src.shape[0]
    epw = e // _NW
    cb = 400
    sub = 80
    nchunk = epw // cb
    mesh = plsc.VectorSubcoreMesh(core_axis_name="c", subcore_axis_name="s")

    nsub = cb // sub

    @functools.partial(
        pl.kernel,
        out_type=jax.ShapeDtypeStruct((e, 64), jnp.float32),
        mesh=mesh,
        scratch_types=[
            pltpu.VMEM((cb,), jnp.int32),
            pltpu.VMEM((cb,), jnp.int32),
            pltpu.VMEM((sub, 128), jnp.float32),
            pltpu.VMEM((sub, 128), jnp.float32),
            pltpu.VMEM((sub, 128), jnp.float32),
            pltpu.VMEM((sub, 128), jnp.float32),
            pltpu.VMEM((cb, 64), jnp.float32),
            pltpu.SemaphoreType.DMA,
            pltpu.SemaphoreType.DMA,
        ],
    )
    def k(pq_hbm, src_hbm, dst_hbm, g_hbm, srcv, dstv,
          prow0, qrow0, prow1, qrow1, gv, sem0, sem1):
        wid = lax.axis_index("s") * _NC + lax.axis_index("c")
        base_w = wid * epw
        pr = (prow0, prow1)
        qr = (qrow0, qrow1)
        sems = (sem0, sem1)

        def fire(s, bsel):
            off = s * sub
            c1 = pltpu.async_copy(
                pq_hbm.at[srcv.at[pl.ds(off, sub)]], pr[bsel], sems[bsel])
            c2 = pltpu.async_copy(
                pq_hbm.at[dstv.at[pl.ds(off, sub)]], qr[bsel], sems[bsel])
            return c1, c2

        @pl.loop(0, nchunk)
        def _chunk(ci):
            base = base_w + ci * cb
            pltpu.sync_copy(src_hbm.at[pl.ds(base, cb)], srcv)
            pltpu.sync_copy(dst_hbm.at[pl.ds(base, cb)], dstv)

            cps = fire(0, 0)
            for s in range(nsub):
                bsel = s % 2
                nxt = cps
                if s + 1 < nsub:
                    nxt = fire(s + 1, (s + 1) % 2)
                cps[0].wait()
                cps[1].wait()
                cps = nxt
                off = s * sub

                @pl.loop(0, sub)
                def _row(r, _off=off, _p=pr[bsel], _q=qr[bsel]):
                    for c in range(4):
                        sl = pl.ds(c * 16, 16)
                        gv[_off + r, sl] = _p[r, sl] \
                            + _q[r, pl.ds(64 + c * 16, 16)]

            pltpu.sync_copy(gv, g_hbm.at[pl.ds(base, cb)])

    return k(pq, src, dst)


def _tc_edge_mlp(g, ef, w0c, b0, w1, b1, w2, b2, w3, b3):
    e = g.shape[0]
    eb = 2000

    def body(g_ref, ef_ref, w0c_ref, b0_ref, w1_ref, b1_ref, w2_ref, b2_ref,
             w3_ref, b3_ref, o_ref):
        h = g_ref[...] + jnp.dot(ef_ref[...], w0c_ref[...],
                                 preferred_element_type=jnp.float32)
        h = jnp.maximum(h + b0_ref[...], 0.0)
        h = jnp.maximum(jnp.dot(h, w1_ref[...],
                                preferred_element_type=jnp.float32)
                        + b1_ref[...], 0.0)
        h = jnp.maximum(jnp.dot(h, w2_ref[...],
                                preferred_element_type=jnp.float32)
                        + b2_ref[...], 0.0)
        t = jnp.dot(h, w3_ref[...], preferred_element_type=jnp.float32) \
            + b3_ref[...]
        kk = jax.nn.sigmoid(t[:, 0:1])
        f12 = t[:, 1:25] * kk
        f3 = t[:, 25:37] * kk
        f4 = t[:, 37:49] * kk
        ones = jnp.ones((eb, 1), jnp.float32)
        z7 = jnp.zeros((eb, 7), jnp.float32)
        z4 = jnp.zeros((eb, 4), jnp.float32)
        z64 = jnp.zeros((eb, 64), jnp.float32)
        o_ref[...] = jnp.concatenate(
            [ones, f12, z7, f3, z4, f4, z4, z64], axis=1)

    wspec = lambda shape: pl.BlockSpec(shape, lambda i: (0, 0))
    return pl.pallas_call(
        body,
        grid=(e // eb,),
        in_specs=[
            pl.BlockSpec((eb, 64), lambda i: (i, 0)),
            pl.BlockSpec((eb, 16), lambda i: (i, 0)),
            wspec((16, 64)), wspec((1, 64)),
            wspec((64, 64)), wspec((1, 64)),
            wspec((64, 64)), wspec((1, 64)),
            wspec((64, 49)), wspec((1, 49)),
        ],
        out_specs=pl.BlockSpec((eb, 128), lambda i: (i, 0)),
        out_shape=jax.ShapeDtypeStruct((e, 128), jnp.float32),
    )(g, ef, w0c, b0, w1, b1, w2, b2, w3, b3)


def _sc_segment_reduce(f, dst):
    e = dst.shape[0]
    npw = 320
    npad = _NW * npw
    dc = 1600
    ndc = e // dc
    pb = 512
    cap = pb + 64
    mesh = plsc.VectorSubcoreMesh(core_axis_name="c", subcore_axis_name="s")

    @functools.partial(
        pl.kernel,
        out_type=jax.ShapeDtypeStruct((npad, 64), jnp.float32),
        mesh=mesh,
        scratch_types=[
            pltpu.VMEM((dc,), jnp.int32),
            pltpu.VMEM((dc,), jnp.int32),
            pltpu.VMEM((cap,), jnp.int32),
            pltpu.VMEM((cap,), jnp.int32),
            pltpu.VMEM((pb, 128), jnp.float32),
            pltpu.VMEM((npw, 64), jnp.float32),
            pltpu.SemaphoreType.DMA,
            pltpu.SemaphoreType.DMA,
            pltpu.SemaphoreType.DMA,
        ],
        compiler_params=pltpu.CompilerParams(needs_layout_passes=False),
    )
    def k(f_hbm, dst_hbm, r_hbm, dstv0, dstv1, idb, ddb, rows, acc,
          semA, semB, semG):
        wid = lax.axis_index("s") * _NC + lax.axis_index("c")
        lo = wid * npw

        @pl.loop(0, npw)
        def _init(r):
            acc[r, pl.ds(0, 16)] = jnp.zeros((16,), jnp.float32)
            acc[r, pl.ds(16, 16)] = jnp.zeros((16,), jnp.float32)
            acc[r, pl.ds(32, 16)] = jnp.full((16,), _BIG, jnp.float32)
            acc[r, pl.ds(48, 16)] = jnp.full((16,), -_BIG, jnp.float32)

        @pl.loop(0, cap // 16)
        def _initb(i):
            idb[pl.ds(i * 16, 16)] = jnp.zeros((16,), jnp.int32)

        def flush(m):
            cps = [pltpu.async_copy(
                f_hbm.at[idb.at[pl.ds(t * 128, 128)]],
                rows.at[pl.ds(t * 128, 128)], semG) for t in range(4)]
            for cp in cps:
                cp.wait()

            @pl.loop(0, m)
            def _rmw(j):
                dl = ddb[pl.ds(j, 16)][0] - lo
                acc[dl, pl.ds(0, 16)] = acc[dl, pl.ds(0, 16)] \
                    + rows[j, pl.ds(0, 16)]
                acc[dl, pl.ds(16, 16)] = acc[dl, pl.ds(16, 16)] \
                    + rows[j, pl.ds(16, 16)]
                acc[dl, pl.ds(32, 16)] = jnp.minimum(
                    acc[dl, pl.ds(32, 16)], rows[j, pl.ds(32, 16)])
                acc[dl, pl.ds(48, 16)] = jnp.maximum(
                    acc[dl, pl.ds(48, 16)], rows[j, pl.ds(48, 16)])

        def append16(d16, eid, m, cnt):
            plsc.store_compressed(idb.at[pl.ds(cnt, 16)], eid, mask=m)
            plsc.store_compressed(ddb.at[pl.ds(cnt, 16)], d16, mask=m)
            return cnt + jnp.sum(m.astype(jnp.int32))

        def scan_buf(buf, ci, cnt):
            def scan32(i, cnt):
                d16a = buf[pl.ds(i * 32, 16)]
                d16b = buf[pl.ds(i * 32 + 16, 16)]
                base = ci * dc + i * 32 + lax.iota(jnp.int32, 16)
                ma = (d16a >= lo) & (d16a < lo + npw)
                mb = (d16b >= lo) & (d16b < lo + npw)
                cnt = append16(d16a, base, ma, cnt)
                cnt = append16(d16b, base + 16, mb, cnt)

                @pl.when(cnt >= pb)
                def _do_flush():
                    flush(pb)
                    idb[pl.ds(0, 16)] = idb[pl.ds(pb, 16)]
                    idb[pl.ds(16, 16)] = idb[pl.ds(pb + 16, 16)]
                    ddb[pl.ds(0, 16)] = ddb[pl.ds(pb, 16)]
                    ddb[pl.ds(16, 16)] = ddb[pl.ds(pb + 16, 16)]

                return jnp.where(cnt >= pb, cnt - pb, cnt)

            return lax.fori_loop(0, dc // 32, scan32, cnt)

        def fire_scan(ci, buf, sem):
            pltpu.async_copy(dst_hbm.at[pl.ds(ci * dc, dc)], buf, sem)

        def wait_scan(buf, sem):
            pltpu.make_async_copy(dst_hbm.at[pl.ds(0, dc)], buf, sem).wait()

        fire_scan(0, dstv0, semA)

        def outer(j, cnt):
            fire_scan(2 * j + 1, dstv1, semB)
            wait_scan(dstv0, semA)
            cnt = scan_buf(dstv0, 2 * j, cnt)

            @pl.when(2 * j + 2 < ndc)
            def _prefetch():
                fire_scan(2 * j + 2, dstv0, semA)

            wait_scan(dstv1, semB)
            return scan_buf(dstv1, 2 * j + 1, cnt)

        cnt = lax.fori_loop(0, ndc // 2, outer, jnp.int32(0))
        flush(cnt)
        pltpu.sync_copy(acc, r_hbm.at[pl.ds(lo, npw)])

    return k(f, dst)


def _tc_node_mlp(nf, r, w0, b0, w1, b1, w2, b2, w3, b3):
    n = nf.shape[0]
    bn = 2000

    def body(nf_ref, r_ref, w0_ref, b0_ref, w1_ref, b1_ref, w2_ref, b2_ref,
             w3_ref, b3_ref, o_ref):
        rr = r_ref[...]
        cnt = rr[:, 0:1]
        nf1 = rr[:, 1:13] / jnp.maximum(cnt, 1.0)
        nf2 = rr[:, 13:25]
        mask = cnt > 0.0
        nf3 = jnp.where(mask, rr[:, 32:44], 0.0)
        nf4 = jnp.where(mask, rr[:, 48:60], 0.0)
        x = jnp.concatenate([nf_ref[...], nf1, nf2, nf3, nf4], axis=1)
        h = jnp.maximum(jnp.dot(x, w0_ref[...],
                                preferred_element_type=jnp.float32)
                        + b0_ref[...], 0.0)
        h = jnp.maximum(jnp.dot(h, w1_ref[...],
                                preferred_element_type=jnp.float32)
                        + b1_ref[...], 0.0)
        h = jnp.maximum(jnp.dot(h, w2_ref[...],
                                preferred_element_type=jnp.float32)
                        + b2_ref[...], 0.0)
        o_ref[...] = jnp.dot(h, w3_ref[...],
                             preferred_element_type=jnp.float32) + b3_ref[...]

    wspec = lambda shape: pl.BlockSpec(shape, lambda i: (0, 0))
    return pl.pallas_call(
        body,
        grid=(n // bn,),
        in_specs=[
            pl.BlockSpec((bn, 128), lambda i: (i, 0)),
            pl.BlockSpec((bn, 64), lambda i: (i, 0)),
            wspec((176, 64)), wspec((1, 64)),
            wspec((64, 64)), wspec((1, 64)),
            wspec((64, 64)), wspec((1, 64)),
            wspec((64, 128)), wspec((1, 128)),
        ],
        out_specs=pl.BlockSpec((bn, 128), lambda i: (i, 0)),
        out_shape=jax.ShapeDtypeStruct((n, 128), jnp.float32),
    )(nf, r, w0, b0, w1, b1, w2, b2, w3, b3)


def kernel(nf, ef, edge_index, msg_W0, msg_b0, msg_W1, msg_b1, msg_W2, msg_b2,
           msg_W3, msg_b3, red_W0, red_b0, red_W1, red_b1, red_W2, red_b2,
           red_W3, red_b3):
    src = edge_index[0]
    dst = edge_index[1]
    w0a = msg_W0[:128]
    w0b = msg_W0[128:256]
    w0c = msg_W0[256:]

    pq = _tc_pq(nf, jnp.concatenate([w0a, w0b], axis=1))
    g = _sc_gather_add(pq, src, dst)
    f = _tc_edge_mlp(
        g, ef, w0c, msg_b0.reshape(1, 64), msg_W1, msg_b1.reshape(1, 64),
        msg_W2, msg_b2.reshape(1, 64), msg_W3, msg_b3.reshape(1, 49))
    r = _sc_segment_reduce(f, dst)
    out = _tc_node_mlp(
        nf, r, red_W0, red_b0.reshape(1, 64), red_W1, red_b1.reshape(1, 64),
        red_W2, red_b2.reshape(1, 64), red_W3, red_b3.reshape(1, 128))
    return out


# speedup vs baseline: 3.9427x; 1.0111x over previous
import functools

import jax
import jax.numpy as jnp
from jax import lax
from jax.experimental import pallas as pl
from jax.experimental.pallas import tpu as pltpu
from jax.experimental.pallas import tpu_sc as plsc

_N = 10000
_E = 320000
_NC = 2
_NS = 16
_NW = _NC * _NS
_BIG = 3.0e38


def _tc_pq(nf, w0ab):
    n = nf.shape[0]
    bn = 2000

    def body(nf_ref, w_ref, pq_ref):
        pq_ref[...] = jnp.dot(nf_ref[...], w_ref[...],
                              preferred_element_type=jnp.float32)

    return pl.pallas_call(
        body,
        grid=(n // bn,),
        in_specs=[
            pl.BlockSpec((bn, 128), lambda i: (i, 0)),
            pl.BlockSpec((128, 128), lambda i: (0, 0)),
        ],
        out_specs=pl.BlockSpec((bn, 128), lambda i: (i, 0)),
        out_shape=jax.ShapeDtypeStruct((n, 128), jnp.float32),
    )(nf, w0ab)


def _sc_gather_add(pq, src, dst):
    e = src.shape[0]
    epw = e // _NW
    cb = 400
    sub = 80
    nchunk = epw // cb
    mesh = plsc.VectorSubcoreMesh(core_axis_name="c", subcore_axis_name="s")

    nsub = cb // sub

    @functools.partial(
        pl.kernel,
        out_type=jax.ShapeDtypeStruct((e, 64), jnp.float32),
        mesh=mesh,
        scratch_types=[
            pltpu.VMEM((cb,), jnp.int32),
            pltpu.VMEM((cb,), jnp.int32),
            pltpu.VMEM((sub, 128), jnp.float32),
            pltpu.VMEM((sub, 128), jnp.float32),
            pltpu.VMEM((sub, 128), jnp.float32),
            pltpu.VMEM((sub, 128), jnp.float32),
            pltpu.VMEM((cb, 64), jnp.float32),
            pltpu.SemaphoreType.DMA,
            pltpu.SemaphoreType.DMA,
        ],
    )
    def k(pq_hbm, src_hbm, dst_hbm, g_hbm, srcv, dstv,
          prow0, qrow0, prow1, qrow1, gv, sem0, sem1):
        wid = lax.axis_index("s") * _NC + lax.axis_index("c")
        base_w = wid * epw
        pr = (prow0, prow1)
        qr = (qrow0, qrow1)
        sems = (sem0, sem1)

        def fire(s, bsel):
            off = s * sub
            c1 = pltpu.async_copy(
                pq_hbm.at[srcv.at[pl.ds(off, sub)]], pr[bsel], sems[bsel])
            c2 = pltpu.async_copy(
                pq_hbm.at[dstv.at[pl.ds(off, sub)]], qr[bsel], sems[bsel])
            return c1, c2

        @pl.loop(0, nchunk)
        def _chunk(ci):
            base = base_w + ci * cb
            pltpu.sync_copy(src_hbm.at[pl.ds(base, cb)], srcv)
            pltpu.sync_copy(dst_hbm.at[pl.ds(base, cb)], dstv)

            cps = fire(0, 0)
            for s in range(nsub):
                bsel = s % 2
                nxt = cps
                if s + 1 < nsub:
                    nxt = fire(s + 1, (s + 1) % 2)
                cps[0].wait()
                cps[1].wait()
                cps = nxt
                off = s * sub

                @pl.loop(0, sub)
                def _row(r, _off=off, _p=pr[bsel], _q=qr[bsel]):
                    for c in range(4):
                        sl = pl.ds(c * 16, 16)
                        gv[_off + r, sl] = _p[r, sl] \
                            + _q[r, pl.ds(64 + c * 16, 16)]

            pltpu.sync_copy(gv, g_hbm.at[pl.ds(base, cb)])

    return k(pq, src, dst)


def _tc_edge_mlp(g, ef, w0c, b0, w1, b1, w2, b2, w3, b3):
    e = g.shape[0]
    eb = 2000

    def body(g_ref, ef_ref, w0c_ref, b0_ref, w1_ref, b1_ref, w2_ref, b2_ref,
             w3_ref, b3_ref, o_ref):
        h = g_ref[...] + jnp.dot(ef_ref[...], w0c_ref[...],
                                 preferred_element_type=jnp.float32)
        h = jnp.maximum(h + b0_ref[...], 0.0)
        h = jnp.maximum(jnp.dot(h, w1_ref[...],
                                preferred_element_type=jnp.float32)
                        + b1_ref[...], 0.0)
        h = jnp.maximum(jnp.dot(h, w2_ref[...],
                                preferred_element_type=jnp.float32)
                        + b2_ref[...], 0.0)
        t = jnp.dot(h, w3_ref[...], preferred_element_type=jnp.float32) \
            + b3_ref[...]
        kk = jax.nn.sigmoid(t[:, 0:1])
        f12 = t[:, 1:25] * kk
        f3 = t[:, 25:37] * kk
        f4 = t[:, 37:49] * kk
        ones = jnp.ones((eb, 1), jnp.float32)
        z7 = jnp.zeros((eb, 7), jnp.float32)
        z4 = jnp.zeros((eb, 4), jnp.float32)
        z64 = jnp.zeros((eb, 64), jnp.float32)
        o_ref[...] = jnp.concatenate(
            [ones, f12, z7, f3, z4, f4, z4, z64], axis=1)

    wspec = lambda shape: pl.BlockSpec(shape, lambda i: (0, 0))
    return pl.pallas_call(
        body,
        grid=(e // eb,),
        in_specs=[
            pl.BlockSpec((eb, 64), lambda i: (i, 0)),
            pl.BlockSpec((eb, 16), lambda i: (i, 0)),
            wspec((16, 64)), wspec((1, 64)),
            wspec((64, 64)), wspec((1, 64)),
            wspec((64, 64)), wspec((1, 64)),
            wspec((64, 49)), wspec((1, 49)),
        ],
        out_specs=pl.BlockSpec((eb, 128), lambda i: (i, 0)),
        out_shape=jax.ShapeDtypeStruct((e, 128), jnp.float32),
    )(g, ef, w0c, b0, w1, b1, w2, b2, w3, b3)


def _sc_segment_reduce(f, dst):
    e = dst.shape[0]
    npw = 320
    npad = _NW * npw
    dc = 1600
    ndc = e // dc
    pb = 256
    cap = pb + 64
    mesh = plsc.VectorSubcoreMesh(core_axis_name="c", subcore_axis_name="s")

    @functools.partial(
        pl.kernel,
        out_type=jax.ShapeDtypeStruct((npad, 64), jnp.float32),
        mesh=mesh,
        scratch_types=[
            pltpu.VMEM((dc,), jnp.int32),
            pltpu.VMEM((dc,), jnp.int32),
            pltpu.VMEM((cap,), jnp.int32),
            pltpu.VMEM((cap,), jnp.int32),
            pltpu.VMEM((pb,), jnp.int32),
            pltpu.VMEM((pb + 16,), jnp.int32),
            pltpu.VMEM((pb,), jnp.int32),
            pltpu.VMEM((pb + 16,), jnp.int32),
            pltpu.VMEM((pb, 128), jnp.float32),
            pltpu.VMEM((pb, 128), jnp.float32),
            pltpu.VMEM((npw, 64), jnp.float32),
            pltpu.SemaphoreType.DMA,
            pltpu.SemaphoreType.DMA,
            pltpu.SemaphoreType.DMA,
        ],
        compiler_params=pltpu.CompilerParams(needs_layout_passes=False),
    )
    def k(f_hbm, dst_hbm, r_hbm, dstv0, dstv1, idb, ddb,
          tib0, tdb0, tib1, tdb1, rows0, rows1, acc, semA, semB, semG):
        wid = lax.axis_index("s") * _NC + lax.axis_index("c")
        lo = wid * npw
        sets = ((tib0, tdb0, rows0), (tib1, tdb1, rows1))

        @pl.loop(0, npw)
        def _init(r):
            acc[r, pl.ds(0, 16)] = jnp.zeros((16,), jnp.float32)
            acc[r, pl.ds(16, 16)] = jnp.zeros((16,), jnp.float32)
            acc[r, pl.ds(32, 16)] = jnp.full((16,), _BIG, jnp.float32)
            acc[r, pl.ds(48, 16)] = jnp.full((16,), -_BIG, jnp.float32)

        @pl.loop(0, cap // 16)
        def _initb(i):
            idb[pl.ds(i * 16, 16)] = jnp.zeros((16,), jnp.int32)

        def rmw(rows_r, ddb_r, m):
            @pl.loop(0, m)
            def _rmw(j):
                dl = ddb_r[pl.ds(j, 16)][0] - lo
                acc[dl, pl.ds(0, 16)] = acc[dl, pl.ds(0, 16)] \
                    + rows_r[j, pl.ds(0, 16)]
                acc[dl, pl.ds(16, 16)] = acc[dl, pl.ds(16, 16)] \
                    + rows_r[j, pl.ds(16, 16)]
                acc[dl, pl.ds(32, 16)] = jnp.minimum(
                    acc[dl, pl.ds(32, 16)], rows_r[j, pl.ds(32, 16)])
                acc[dl, pl.ds(48, 16)] = jnp.maximum(
                    acc[dl, pl.ds(48, 16)], rows_r[j, pl.ds(48, 16)])

        def fire_gather(idx_r, rows_r):
            for t in range(pb // 128):
                pltpu.async_copy(
                    f_hbm.at[idx_r.at[pl.ds(t * 128, 128)]],
                    rows_r.at[pl.ds(t * 128, 128)], semG)

        def wait_gather(idx_r, rows_r):
            for t in range(pb // 128):
                pltpu.make_async_copy(
                    f_hbm.at[idx_r.at[pl.ds(t * 128, 128)]],
                    rows_r.at[pl.ds(t * 128, 128)], semG).wait()

        def do_flush(par, fc):
            tib, tdb, rows_c = sets[par]
            o_tib, o_tdb, o_rows = sets[1 - par]

            @pl.when(fc > 0)
            def _consume_prev():
                wait_gather(o_tib, o_rows)
                rmw(o_rows, o_tdb, pb)

            @pl.loop(0, pb // 16)
            def _xfer(i):
                sl = pl.ds(i * 16, 16)
                tib[sl] = idb[sl]
                tdb[sl] = ddb[sl]

            fire_gather(tib, rows_c)
            idb[pl.ds(0, 16)] = idb[pl.ds(pb, 16)]
            idb[pl.ds(16, 16)] = idb[pl.ds(pb + 16, 16)]
            ddb[pl.ds(0, 16)] = ddb[pl.ds(pb, 16)]
            ddb[pl.ds(16, 16)] = ddb[pl.ds(pb + 16, 16)]

        def append16(d16, eid, m, cnt):
            plsc.store_compressed(idb.at[pl.ds(cnt, 16)], eid, mask=m)
            plsc.store_compressed(ddb.at[pl.ds(cnt, 16)], d16, mask=m)
            return cnt + jnp.sum(m.astype(jnp.int32))

        def scan_buf(buf, ci, carry):
            def scan32(i, carry):
                cnt, fc = carry
                d16a = buf[pl.ds(i * 32, 16)]
                d16b = buf[pl.ds(i * 32 + 16, 16)]
                base = ci * dc + i * 32 + lax.iota(jnp.int32, 16)
                ma = (d16a >= lo) & (d16a < lo + npw)
                mb = (d16b >= lo) & (d16b < lo + npw)
                cnt = append16(d16a, base, ma, cnt)
                cnt = append16(d16b, base + 16, mb, cnt)
                full = cnt >= pb

                @pl.when(full & (fc % 2 == 0))
                def _f0():
                    do_flush(0, fc)

                @pl.when(full & (fc % 2 == 1))
                def _f1():
                    do_flush(1, fc)

                return (jnp.where(full, cnt - pb, cnt),
                        jnp.where(full, fc + 1, fc))

            return lax.fori_loop(0, dc // 32, scan32, carry)

        def fire_scan(ci, buf, sem):
            pltpu.async_copy(dst_hbm.at[pl.ds(ci * dc, dc)], buf, sem)

        def wait_scan(buf, sem):
            pltpu.make_async_copy(dst_hbm.at[pl.ds(0, dc)], buf, sem).wait()

        fire_scan(0, dstv0, semA)

        def outer(j, carry):
            fire_scan(2 * j + 1, dstv1, semB)
            wait_scan(dstv0, semA)
            carry = scan_buf(dstv0, 2 * j, carry)

            @pl.when(2 * j + 2 < ndc)
            def _prefetch():
                fire_scan(2 * j + 2, dstv0, semA)

            wait_scan(dstv1, semB)
            return scan_buf(dstv1, 2 * j + 1, carry)

        cnt, fc = lax.fori_loop(0, ndc // 2, outer,
                                (jnp.int32(0), jnp.int32(0)))

        @pl.when(fc % 2 == 1)
        def _drain0():
            wait_gather(tib0, rows0)
            rmw(rows0, tdb0, pb)

        @pl.when((fc > 0) & (fc % 2 == 0))
        def _drain1():
            wait_gather(tib1, rows1)
            rmw(rows1, tdb1, pb)

        fire_gather(idb, rows0)
        wait_gather(idb, rows0)
        rmw(rows0, ddb, cnt)
        pltpu.sync_copy(acc, r_hbm.at[pl.ds(lo, npw)])

    return k(f, dst)


def _tc_node_mlp(nf, r, w0, b0, w1, b1, w2, b2, w3, b3):
    n = nf.shape[0]
    bn = 2000

    def body(nf_ref, r_ref, w0_ref, b0_ref, w1_ref, b1_ref, w2_ref, b2_ref,
             w3_ref, b3_ref, o_ref):
        rr = r_ref[...]
        cnt = rr[:, 0:1]
        nf1 = rr[:, 1:13] / jnp.maximum(cnt, 1.0)
        nf2 = rr[:, 13:25]
        mask = cnt > 0.0
        nf3 = jnp.where(mask, rr[:, 32:44], 0.0)
        nf4 = jnp.where(mask, rr[:, 48:60], 0.0)
        x = jnp.concatenate([nf_ref[...], nf1, nf2, nf3, nf4], axis=1)
        h = jnp.maximum(jnp.dot(x, w0_ref[...],
                                preferred_element_type=jnp.float32)
                        + b0_ref[...], 0.0)
        h = jnp.maximum(jnp.dot(h, w1_ref[...],
                                preferred_element_type=jnp.float32)
                        + b1_ref[...], 0.0)
        h = jnp.maximum(jnp.dot(h, w2_ref[...],
                                preferred_element_type=jnp.float32)
                        + b2_ref[...], 0.0)
        o_ref[...] = jnp.dot(h, w3_ref[...],
                             preferred_element_type=jnp.float32) + b3_ref[...]

    wspec = lambda shape: pl.BlockSpec(shape, lambda i: (0, 0))
    return pl.pallas_call(
        body,
        grid=(n // bn,),
        in_specs=[
            pl.BlockSpec((bn, 128), lambda i: (i, 0)),
            pl.BlockSpec((bn, 64), lambda i: (i, 0)),
            wspec((176, 64)), wspec((1, 64)),
            wspec((64, 64)), wspec((1, 64)),
            wspec((64, 64)), wspec((1, 64)),
            wspec((64, 128)), wspec((1, 128)),
        ],
        out_specs=pl.BlockSpec((bn, 128), lambda i: (i, 0)),
        out_shape=jax.ShapeDtypeStruct((n, 128), jnp.float32),
    )(nf, r, w0, b0, w1, b1, w2, b2, w3, b3)


def kernel(nf, ef, edge_index, msg_W0, msg_b0, msg_W1, msg_b1, msg_W2, msg_b2,
           msg_W3, msg_b3, red_W0, red_b0, red_W1, red_b1, red_W2, red_b2,
           red_W3, red_b3):
    src = edge_index[0]
    dst = edge_index[1]
    w0a = msg_W0[:128]
    w0b = msg_W0[128:256]
    w0c = msg_W0[256:]

    pq = _tc_pq(nf, jnp.concatenate([w0a, w0b], axis=1))
    g = _sc_gather_add(pq, src, dst)
    f = _tc_edge_mlp(
        g, ef, w0c, msg_b0.reshape(1, 64), msg_W1, msg_b1.reshape(1, 64),
        msg_W2, msg_b2.reshape(1, 64), msg_W3, msg_b3.reshape(1, 49))
    r = _sc_segment_reduce(f, dst)
    out = _tc_node_mlp(
        nf, r, red_W0, red_b0.reshape(1, 64), red_W1, red_b1.reshape(1, 64),
        red_W2, red_b2.reshape(1, 64), red_W3, red_b3.reshape(1, 128))
    return out
